# baseline (speedup 1.0000x reference)
import functools

import jax
import jax.numpy as jnp
from jax import lax
from jax.experimental import pallas as pl
from jax.experimental.pallas import tpu as pltpu
from jax.experimental.pallas import tpu_sc as plsc

_S = 2048
_H = 1024
_V = 50000

_NC = 2
_NS = 16
_NW = _NC * _NS
_BPW = _S // _NW

_TN = 1024


def _sc_gather_body(table_hbm, idx_hbm, out_hbm, idx_v, rows_v, sem):
    wid = lax.axis_index("s") * _NC + lax.axis_index("c")
    base = wid * _BPW
    pltpu.sync_copy(idx_hbm.at[pl.ds(base, _BPW)], idx_v)
    pltpu.async_copy(table_hbm.at[idx_v], rows_v, sem).wait()
    pltpu.sync_copy(rows_v, out_hbm.at[pl.ds(base, _BPW)])


def _sc_gather(table, idx):
    run = functools.partial(
        pl.kernel,
        mesh=plsc.VectorSubcoreMesh(core_axis_name="c", subcore_axis_name="s"),
        out_type=jax.ShapeDtypeStruct((_S, _H), jnp.float32),
        scratch_types=[
            pltpu.VMEM((_BPW,), jnp.int32),
            pltpu.VMEM((_BPW, _H), jnp.float32),
            pltpu.SemaphoreType.DMA,
        ],
    )(_sc_gather_body)
    return run(table, idx)


def _mm_body(h_ref, w_ref, b_ref, o_ref):
    w = w_ref[...].astype(jnp.bfloat16)
    o_ref[...] = (
        jnp.dot(h_ref[...], w, preferred_element_type=jnp.float32) + b_ref[...]
    )


def _matmul(h_bf16, w, b2d):
    return pl.pallas_call(
        _mm_body,
        grid=(pl.cdiv(_V, _TN),),
        in_specs=[
            pl.BlockSpec((_S, _H), lambda j: (0, 0)),
            pl.BlockSpec((_H, _TN), lambda j: (0, j)),
            pl.BlockSpec((1, _TN), lambda j: (0, j)),
        ],
        out_specs=pl.BlockSpec((_S, _TN), lambda j: (0, j)),
        out_shape=jax.ShapeDtypeStruct((_S, _V), jnp.float32),
        compiler_params=pltpu.CompilerParams(
            dimension_semantics=("arbitrary",),
        ),
    )(h_bf16, w, b2d)


def kernel(inputs, embed_table, W, b):
    idx = inputs.reshape(_S).astype(jnp.int32)
    hidden = _sc_gather(embed_table, idx)
    logits = _matmul(hidden.astype(jnp.bfloat16), W, b.reshape(1, _V))
    return (hidden.reshape(1, _S, _H), logits.reshape(1, _S, _V))
